# baseline (speedup 1.0000x reference)
import jax
import jax.numpy as jnp
from jax.experimental import pallas as pl

D_MODEL = 2048
N_EXPERTS = 16
CAPACITY_FACTOR = 1.0
EPSILON = 1e-06
S_BLK = 512


def _gate_kernel(x_ref, w_ref, b_ref, o_ref):
    B, S, D = x_ref.shape
    x = x_ref[...].reshape(B * S, D)
    logits = jnp.dot(x, w_ref[...], preferred_element_type=jnp.float32) + b_ref[...]

    m1 = jnp.max(logits, axis=-1, keepdims=True)
    hot1 = logits == m1
    m2 = jnp.max(jnp.where(hot1, -jnp.inf, logits), axis=-1, keepdims=True)
    hot = logits >= m2

    e = jnp.exp(logits - m1)
    rowsum = jnp.sum(e, axis=-1, keepdims=True)
    masked = jnp.where(hot, e / rowsum, 0.0).reshape(B, S, N_EXPERTS)

    denom = jnp.sum(masked, axis=0, keepdims=True) + EPSILON
    capacity = int(CAPACITY_FACTOR * B)
    o_ref[...] = masked * (capacity / denom)


def kernel(X, W, b):
    B, S, D = X.shape
    return pl.pallas_call(
        _gate_kernel,
        grid=(S // S_BLK,),
        in_specs=[
            pl.BlockSpec((B, S_BLK, D), lambda i: (0, i, 0)),
            pl.BlockSpec((D, N_EXPERTS), lambda i: (0, 0)),
            pl.BlockSpec((1, N_EXPERTS), lambda i: (0, 0)),
        ],
        out_specs=pl.BlockSpec((B, S_BLK, N_EXPERTS), lambda i: (0, i, 0)),
        out_shape=jax.ShapeDtypeStruct((B, S, N_EXPERTS), jnp.float32),
    )(X, W, b.reshape(1, N_EXPERTS))


# speedup vs baseline: 1.0344x; 1.0344x over previous
import jax
import jax.numpy as jnp
from jax.experimental import pallas as pl

D_MODEL = 2048
N_EXPERTS = 16
CAPACITY_FACTOR = 1.0
EPSILON = 1e-06
S_BLK = 256


def _gate_kernel(x_ref, w_ref, b_ref, o_ref):
    B, S, D = x_ref.shape
    w = w_ref[...]
    bias = b_ref[...]

    masked = []
    for b in range(B):
        logits = jnp.dot(x_ref[b], w, preferred_element_type=jnp.float32) + bias

        m1 = jnp.max(logits, axis=-1, keepdims=True)
        m2 = jnp.max(jnp.where(logits == m1, -jnp.inf, logits), axis=-1, keepdims=True)
        hot = logits >= m2

        e = jnp.exp(logits - m1)
        rowsum = jnp.sum(e, axis=-1, keepdims=True)
        masked.append(jnp.where(hot, e / rowsum, 0.0))

    denom = masked[0]
    for b in range(1, B):
        denom = denom + masked[b]
    scale = CAPACITY_FACTOR * B / (denom + EPSILON)
    for b in range(B):
        o_ref[b] = masked[b] * scale


def kernel(X, W, b):
    B, S, D = X.shape
    return pl.pallas_call(
        _gate_kernel,
        grid=(S // S_BLK,),
        in_specs=[
            pl.BlockSpec((B, S_BLK, D), lambda i: (0, i, 0)),
            pl.BlockSpec((D, N_EXPERTS), lambda i: (0, 0)),
            pl.BlockSpec((1, N_EXPERTS), lambda i: (0, 0)),
        ],
        out_specs=pl.BlockSpec((B, S_BLK, N_EXPERTS), lambda i: (0, i, 0)),
        out_shape=jax.ShapeDtypeStruct((B, S, N_EXPERTS), jnp.float32),
    )(X, W, b.reshape(1, N_EXPERTS))
